# baseline (speedup 1.0000x reference)
import functools

import jax
import jax.numpy as jnp
from jax import lax
from jax.experimental import pallas as pl
from jax.experimental.pallas import tpu as pltpu
from jax.experimental.pallas import tpu_sc as plsc

NUM_EMB = 1000000
DIM = 64

NC = 2
NS = 16
NW = NC * NS

B_TOTAL = 16384 * 50
CHUNK = 128
B_PER_W = B_TOTAL // NW
NCHUNK = B_PER_W // CHUNK


NB = 8
NOUT = NCHUNK // NB


def _emb_body(table_hbm, idx_hbm, out_hbm, idx_v, rows_v, gsem):
    wid = lax.axis_index("s") * NC + lax.axis_index("c")
    pltpu.sync_copy(idx_hbm.at[wid], idx_v)

    for b in range(NB):
        pltpu.async_copy(table_hbm.at[idx_v.at[b]], rows_v.at[b], gsem)

    def outer(o, carry):
        for b in range(NB):
            j = o * NB + b
            pltpu.make_async_copy(table_hbm.at[idx_v.at[j]], rows_v.at[b], gsem).wait()
            pltpu.sync_copy(rows_v.at[b], out_hbm.at[wid, j])
            pltpu.async_copy(table_hbm.at[idx_v.at[j + NB]], rows_v.at[b], gsem)
        return carry

    lax.fori_loop(0, NOUT - 1, outer, 0)

    for b in range(NB):
        j = (NOUT - 1) * NB + b
        pltpu.make_async_copy(table_hbm.at[idx_v.at[j]], rows_v.at[b], gsem).wait()
        pltpu.sync_copy(rows_v.at[b], out_hbm.at[wid, j])


@jax.jit
def _emb_lookup(weight, idx):
    return pl.kernel(
        _emb_body,
        out_type=jax.ShapeDtypeStruct((NW, NCHUNK, CHUNK, DIM), jnp.float32),
        mesh=plsc.VectorSubcoreMesh(core_axis_name="c", subcore_axis_name="s"),
        compiler_params=pltpu.CompilerParams(use_tc_tiling_on_sc=False),
        scratch_types=[
            pltpu.VMEM((NCHUNK, CHUNK), jnp.int32),
            pltpu.VMEM((NB, CHUNK, DIM), jnp.float32),
            pltpu.SemaphoreType.DMA,
        ],
    )(weight, idx)


def kernel(token_ids, weight):
    b, s = token_ids.shape
    idx = token_ids.reshape(NW, NCHUNK, CHUNK).astype(jnp.int32)
    out = _emb_lookup(weight, idx)
    return out.reshape(b, s, DIM)


# speedup vs baseline: 1.0024x; 1.0024x over previous
import functools

import jax
import jax.numpy as jnp
from jax import lax
from jax.experimental import pallas as pl
from jax.experimental.pallas import tpu as pltpu
from jax.experimental.pallas import tpu_sc as plsc

NUM_EMB = 1000000
DIM = 64

NC = 2
NS = 16
NW = NC * NS

B_TOTAL = 16384 * 50
CHUNK = 128
B_PER_W = B_TOTAL // NW
NCHUNK = B_PER_W // CHUNK
NB = 8
LAG = 4
NOUT = NCHUNK // NB


def _emb_body(table_hbm, idx_hbm, out_hbm, idx_v, rows_v, gsem, ssem):
    wid = lax.axis_index("s") * NC + lax.axis_index("c")
    pltpu.sync_copy(idx_hbm.at[wid], idx_v)

    def gather(j, b):
        pltpu.async_copy(table_hbm.at[idx_v.at[j]], rows_v.at[b], gsem)

    def wait_gather(j, b):
        pltpu.make_async_copy(table_hbm.at[idx_v.at[j]], rows_v.at[b], gsem).wait()

    def store(j, b):
        pltpu.async_copy(rows_v.at[b], out_hbm.at[wid, j], ssem)

    def wait_store(j, b):
        pltpu.make_async_copy(rows_v.at[b], out_hbm.at[wid, j], ssem).wait()

    for b in range(NB):
        gather(b, b)

    def outer(o, carry):
        for b in range(NB):
            j = o * NB + b
            wait_gather(j, b)
            store(j, b)
            g = j + NB - LAG
            bg = (b + NB - LAG) % NB

            @pl.when(jnp.logical_and(g >= NB, g < NCHUNK))
            def _():
                wait_store(g - NB, bg)
                gather(g, bg)

        return carry

    lax.fori_loop(0, NOUT, outer, 0)

    for i in range(NB):
        j = NCHUNK - NB + i
        wait_store(j, j % NB)


@jax.jit
def _emb_lookup(weight, idx):
    return pl.kernel(
        _emb_body,
        out_type=jax.ShapeDtypeStruct((NW, NCHUNK, CHUNK, DIM), jnp.float32),
        mesh=plsc.VectorSubcoreMesh(core_axis_name="c", subcore_axis_name="s"),
        compiler_params=pltpu.CompilerParams(use_tc_tiling_on_sc=False),
        scratch_types=[
            pltpu.VMEM((NCHUNK, CHUNK), jnp.int32),
            pltpu.VMEM((NB, CHUNK, DIM), jnp.float32),
            pltpu.SemaphoreType.DMA,
            pltpu.SemaphoreType.DMA,
        ],
    )(weight, idx)


def kernel(token_ids, weight):
    b, s = token_ids.shape
    idx = token_ids.reshape(NW, NCHUNK, CHUNK).astype(jnp.int32)
    out = _emb_lookup(weight, idx)
    return out.reshape(b, s, DIM)
